# baseline (speedup 1.0000x reference)
import jax
import jax.numpy as jnp
from jax import lax
from jax.experimental import pallas as pl
from jax.experimental.pallas import tpu as pltpu
from jax.experimental.pallas import tpu_sc as plsc

VOCAB = 1000000
MAX_LEN = 200
D = 64
B = 1024
T = 200

NC = 2
NS = 16
NW = NC * NS
CH = 40
CPW = (B * T) // (NW * CH)
NBUF = 4
LANES = 16
VPR = D // LANES


def _sc_body(tok2, idxd, par, pos, out, idxd_v, par_v, pos_v,
             i0, i1, i2, i3, o0, o1, o2, o3,
             g0, g1, g2, g3, s0, s1, s2, s3):
    ibufs = (i0, i1, i2, i3)
    obufs = (o0, o1, o2, o3)
    gsem = (g0, g1, g2, g3)
    ssem = (s0, s1, s2, s3)
    wid = lax.axis_index("s") * NC + lax.axis_index("c")
    row0 = wid * CPW
    out0 = wid * CPW * CH

    pltpu.sync_copy(idxd.at[pl.ds(row0, CPW)], idxd_v)
    pltpu.sync_copy(par.at[pl.ds(row0, CPW)], par_v)
    pltpu.sync_copy(pos, pos_v)

    def gather(s, b):
        pltpu.async_copy(tok2.at[idxd_v.at[s]], ibufs[b], gsem[b])

    def wait_gather(s, b):
        pltpu.make_async_copy(tok2.at[idxd_v.at[s]], ibufs[b], gsem[b]).wait()

    def store(s, b):
        pltpu.async_copy(obufs[b], out.at[pl.ds(out0 + s * CH, CH)], ssem[b])

    def wait_store(s, b):
        pltpu.make_async_copy(
            obufs[b], out.at[pl.ds(out0 + s * CH, CH)], ssem[b]).wait()

    for s in range(NBUF - 1):
        gather(s, s)

    def group(i, carry):
        g = i * NBUF
        for b in range(NBUF):
            s = g + b
            t = s + NBUF - 1
            tb = (b + NBUF - 1) % NBUF

            @pl.when(t < CPW)
            def _():
                gather(t, tb)

            wait_gather(s, b)

            @pl.when(s >= NBUF)
            def _():
                wait_store(s - NBUF, b)

            off = lax.rem(s, T // CH) * CH

            for rb in range(0, CH, LANES):
                hv = par_v[s, pl.ds(rb, LANES)]
                for j in range(min(LANES, CH - rb)):
                    r = rb + j
                    half = hv[j]
                    for v in range(VPR):
                        dst = pl.ds(v * LANES, LANES)
                        src = pl.ds(half + v * LANES, LANES)
                        obufs[b][r, dst] = (
                            ibufs[b][r, src] + pos_v[off + r, dst])

            store(s, b)
        return carry

    lax.fori_loop(0, CPW // NBUF, group, 0)

    for s in range(CPW - NBUF, CPW):
        wait_store(s, s % NBUF)


def kernel(input_ids, token_emb, pos_emb):
    ids = input_ids.reshape(NW * CPW, CH).astype(jnp.int32)
    tok2 = token_emb.reshape(VOCAB // 2, 2 * D)
    idxd = ids // 2
    par = jnp.pad((ids % 2) * D, ((0, 0), (0, 48 - CH)))
    mesh = plsc.VectorSubcoreMesh(core_axis_name="c", subcore_axis_name="s")
    out = pl.kernel(
        _sc_body,
        out_type=jax.ShapeDtypeStruct((B * T, D), jnp.float32),
        mesh=mesh,
        compiler_params=pltpu.CompilerParams(use_tc_tiling_on_sc=True),
        scratch_types=[
            pltpu.VMEM((CPW, CH), jnp.int32),
            pltpu.VMEM((CPW, 48), jnp.int32),
            pltpu.VMEM((MAX_LEN, D), jnp.float32),
        ] + [pltpu.VMEM((CH, 2 * D), jnp.float32) for _ in range(NBUF)]
          + [pltpu.VMEM((CH, D), jnp.float32) for _ in range(NBUF)]
          + [pltpu.SemaphoreType.DMA for _ in range(2 * NBUF)],
    )(tok2, idxd, par, pos_emb)
    return out.reshape(B, T, D)


# speedup vs baseline: 1.6083x; 1.6083x over previous
import functools

import jax
import jax.numpy as jnp
from jax import lax
from jax.experimental import pallas as pl
from jax.experimental.pallas import tpu as pltpu
from jax.experimental.pallas import tpu_sc as plsc

VOCAB = 1000000
MAX_LEN = 200
D = 64
B = 1024
T = 200

NC = 2
NS = 16
NW = NC * NS
CH = 40
CPW = (B * T) // (NW * CH)
NBUF = 4
LANES = 16
VPR = D // LANES

VBLK = 4096


def _pack_body(tt_ref, out_ref):
    out_ref[:, pl.ds(0, D)] = tt_ref[...].T


def _pack_table(token_t):
    grid = pl.cdiv(VOCAB, VBLK)
    return pl.pallas_call(
        _pack_body,
        grid=(grid,),
        in_specs=[pl.BlockSpec((D, VBLK), lambda j: (0, j))],
        out_specs=pl.BlockSpec((VBLK, 2 * D), lambda j: (j, 0)),
        out_shape=jax.ShapeDtypeStruct((VOCAB, 2 * D), jnp.float32),
    )(token_t)


def _sc_body(tok, idx, pos, out, idx_v, pos_v, b0, b1, b2, b3,
             o0, o1, o2, o3, g0, g1, g2, g3, s0, s1, s2, s3):
    bufs = (b0, b1, b2, b3)
    obufs = (o0, o1, o2, o3)
    gsem = (g0, g1, g2, g3)
    ssem = (s0, s1, s2, s3)
    wid = lax.axis_index("s") * NC + lax.axis_index("c")
    row0 = wid * CPW
    out0 = wid * CPW * CH

    pltpu.sync_copy(idx.at[pl.ds(row0, CPW)], idx_v)
    pltpu.sync_copy(pos, pos_v)

    def gather(s, b):
        pltpu.async_copy(tok.at[idx_v.at[s]], bufs[b], gsem[b])

    def wait_gather(s, b):
        pltpu.make_async_copy(tok.at[idx_v.at[s]], bufs[b], gsem[b]).wait()

    def store(s, b):
        pltpu.async_copy(obufs[b], out.at[pl.ds(out0 + s * CH, CH)], ssem[b])

    def wait_store(s, b):
        pltpu.make_async_copy(
            obufs[b], out.at[pl.ds(out0 + s * CH, CH)], ssem[b]).wait()

    for s in range(NBUF - 1):
        gather(s, s)

    def group(i, carry):
        g = i * NBUF
        for b in range(NBUF):
            s = g + b
            wait_gather(s, b)

            off = lax.rem(s, T // CH) * CH

            def addpos(r, c, _b=b, _off=off):
                for v in range(VPR):
                    sl = pl.ds(v * LANES, LANES)
                    obufs[_b][r, sl] = bufs[_b][r, sl] + pos_v[_off + r, sl]
                return c
            lax.fori_loop(0, CH, addpos, 0, unroll=2)

            t = s + NBUF - 1
            tb = (b + NBUF - 1) % NBUF

            @pl.when(t < CPW)
            def _():
                @pl.when(t >= NBUF)
                def _():
                    wait_store(t - NBUF, tb)
                gather(t, tb)

            store(s, b)
        return carry

    lax.fori_loop(0, CPW // NBUF, group, 0)

    for s in range(CPW - NBUF, CPW):
        wait_store(s, s % NBUF)


def kernel(input_ids, token_emb, pos_emb):
    ids = input_ids.reshape(NW * CPW, CH).astype(jnp.int32)
    tok = _pack_table(token_emb.T)
    mesh = plsc.VectorSubcoreMesh(core_axis_name="c", subcore_axis_name="s")
    out = pl.kernel(
        _sc_body,
        out_type=jax.ShapeDtypeStruct((B * T, D), jnp.float32),
        mesh=mesh,
        compiler_params=pltpu.CompilerParams(use_tc_tiling_on_sc=True),
        scratch_types=[
            pltpu.VMEM((CPW, CH), jnp.int32),
            pltpu.VMEM((MAX_LEN, D), jnp.float32),
        ] + [pltpu.VMEM((CH, 2 * D), jnp.float32) for _ in range(NBUF)]
          + [pltpu.VMEM((CH, D), jnp.float32) for _ in range(NBUF)]
          + [pltpu.SemaphoreType.DMA for _ in range(2 * NBUF)],
    )(tok, ids, pos_emb)
    return out.reshape(B, T, D)
